# baseline (speedup 1.0000x reference)
import functools

import jax
import jax.numpy as jnp
from jax.experimental import pallas as pl

B = 16384
C = 10
HALF = 8192
BB = 4096


def _body(tp_ref, pp_ref, bayes_ref, o_ref):
    step = pl.program_id(0)

    bayes = bayes_ref[...]

    m = [(bayes == r).astype(jnp.float32) for r in range(C)]

    acc_neg = jnp.zeros((BB,), jnp.float32)
    total = jnp.zeros((BB,), jnp.float32)
    for c in range(C):
        tc = tp_ref[:, c, :]
        q_c = tc[0] * m[0]
        for r in range(1, C):
            q_c = q_c + tc[r] * m[r]
        sgn = 1.0 - 2.0 * pp_ref[c, :].astype(jnp.float32)
        e_c = jnp.exp(q_c * sgn)
        acc_neg = acc_neg + e_c * (0.5 * (1.0 + sgn))
        total = total + e_c

    acc_pos = total - acc_neg
    part = jnp.sum(jnp.log1p(acc_neg * acc_pos), keepdims=True) * (1.0 / B)

    @pl.when(step == 0)
    def _():
        o_ref[...] = jnp.zeros_like(o_ref)

    o_ref[...] += part.reshape(1, 1)


def _half(tp, pp, bayes):
    return pl.pallas_call(
        _body,
        grid=(HALF // BB,),
        in_specs=[
            pl.BlockSpec((C, C, BB), lambda i: (0, 0, i)),
            pl.BlockSpec((C, BB), lambda i: (0, i)),
            pl.BlockSpec((BB,), lambda i: (i,)),
        ],
        out_specs=pl.BlockSpec((1, 1), lambda i: (0, 0)),
        out_shape=jax.ShapeDtypeStruct((1, 1), jnp.float32),
    )(tp, pp, bayes)


@jax.jit
def kernel(T, bayes, partial):
    tp = jnp.transpose(T, (1, 2, 0))
    pp = partial.T
    parts = [
        _half(
            jax.lax.slice_in_dim(tp, h * HALF, (h + 1) * HALF, axis=2),
            jax.lax.slice_in_dim(pp, h * HALF, (h + 1) * HALF, axis=1),
            jax.lax.slice_in_dim(bayes, h * HALF, (h + 1) * HALF, axis=0),
        )
        for h in range(B // HALF)
    ]
    return sum(p[0, 0] for p in parts)


# speedup vs baseline: 3.7328x; 3.7328x over previous
import functools

import jax
import jax.numpy as jnp
from jax.experimental import pallas as pl

B = 16384
C = 10
BB = 8192
GRID = B // BB


def _body(tp_ref, pp_ref, bayes_ref, o_ref):
    step = pl.program_id(0)

    bayes = bayes_ref[...]

    m = [(bayes == r).astype(jnp.float32) for r in range(C)]

    acc_neg = jnp.zeros((BB,), jnp.float32)
    total = jnp.zeros((BB,), jnp.float32)
    for c in range(C):
        tc = tp_ref[:, c, :]
        q_c = tc[0] * m[0]
        for r in range(1, C):
            q_c = q_c + tc[r] * m[r]
        sgn = 1.0 - 2.0 * pp_ref[c, :].astype(jnp.float32)
        e_c = jnp.exp(q_c * sgn)
        acc_neg = acc_neg + e_c * (0.5 * (1.0 + sgn))
        total = total + e_c

    acc_pos = total - acc_neg
    part = jnp.sum(jnp.log1p(acc_neg * acc_pos), keepdims=True) * (1.0 / B)

    @pl.when(step == 0)
    def _():
        o_ref[...] = jnp.zeros_like(o_ref)

    o_ref[...] += part.reshape(1, 1)


@jax.jit
def kernel(T, bayes, partial):
    tp = jnp.transpose(T, (1, 2, 0))
    pp = partial.T
    out = pl.pallas_call(
        _body,
        grid=(GRID,),
        in_specs=[
            pl.BlockSpec((C, C, BB), lambda i: (0, 0, i)),
            pl.BlockSpec((C, BB), lambda i: (0, i)),
            pl.BlockSpec((BB,), lambda i: (i,)),
        ],
        out_specs=pl.BlockSpec((1, 1), lambda i: (0, 0)),
        out_shape=jax.ShapeDtypeStruct((1, 1), jnp.float32),
    )(tp, pp, bayes)
    return out[0, 0]


# speedup vs baseline: 3.7898x; 1.0153x over previous
import functools

import jax
import jax.numpy as jnp
from jax.experimental import pallas as pl

B = 16384
C = 10
BB = 8192
GRID = B // BB


def _body(tp_ref, pp_ref, bayes_ref, o_ref):
    step = pl.program_id(0)

    bayes = bayes_ref[...]

    m = [(bayes == r).astype(jnp.float32) for r in range(C)]

    acc_neg = jnp.zeros((BB,), jnp.float32)
    total = jnp.zeros((BB,), jnp.float32)
    for cc in range(C // 2):
        c0, c1 = 2 * cc, 2 * cc + 1
        ta = tp_ref[:, c0, :]
        tb = tp_ref[:, c1, :]
        q_a = ta[0] * m[0]
        q_b = tb[0] * m[0]
        for r in range(1, C):
            q_a = q_a + ta[r] * m[r]
            q_b = q_b + tb[r] * m[r]
        for c, q_c in ((c0, q_a), (c1, q_b)):
            sgn = 1.0 - 2.0 * pp_ref[c, :].astype(jnp.float32)
            e_c = jnp.exp(q_c * sgn)
            acc_neg = acc_neg + e_c * (0.5 * (1.0 + sgn))
            total = total + e_c

    acc_pos = total - acc_neg
    part = jnp.sum(jnp.log1p(acc_neg * acc_pos), keepdims=True) * (1.0 / B)

    @pl.when(step == 0)
    def _():
        o_ref[...] = jnp.zeros_like(o_ref)

    o_ref[...] += part.reshape(1, 1)


@jax.jit
def kernel(T, bayes, partial):
    tp = jnp.transpose(T, (1, 2, 0))
    pp = partial.T
    out = pl.pallas_call(
        _body,
        grid=(GRID,),
        in_specs=[
            pl.BlockSpec((C, C, BB), lambda i: (0, 0, i)),
            pl.BlockSpec((C, BB), lambda i: (0, i)),
            pl.BlockSpec((BB,), lambda i: (i,)),
        ],
        out_specs=pl.BlockSpec((1, 1), lambda i: (0, 0)),
        out_shape=jax.ShapeDtypeStruct((1, 1), jnp.float32),
    )(tp, pp, bayes)
    return out[0, 0]
